# baseline (speedup 1.0000x reference)
import functools

import jax
import jax.numpy as jnp
from jax import lax
from jax.experimental import pallas as pl
from jax.experimental.pallas import tpu as pltpu
from jax.experimental.pallas import tpu_sc as plsc

N_Q = 4096
N_F = 20000
F_PAD = 20480
NW = 32
FBS = 512
QB = 4096


def _sum3_mean(a, b, c):
    return (a + b) + c


def _sum3_sq(a, b, c):
    return (a + c) + b


def _sum3_dot(a, b, c):
    return (a + c) + b


def _bf(x):
    return x.astype(jnp.bfloat16).astype(jnp.float32)


def _sc_gather_rows(table, idx):
    B = idx.shape[0]
    assert B % (8 * NW) == 0
    bpw = B // NW
    mesh = plsc.VectorSubcoreMesh(core_axis_name="c", subcore_axis_name="s")

    @functools.partial(
        pl.kernel,
        mesh=mesh,
        out_type=jax.ShapeDtypeStruct((B, 16), table.dtype),
        compiler_params=pltpu.CompilerParams(use_tc_tiling_on_sc=False),
        scratch_types=[
            pltpu.VMEM((bpw,), jnp.int32),
            pltpu.VMEM((bpw, 16), table.dtype),
            pltpu.SemaphoreType.DMA,
        ],
    )
    def k(table_hbm, idx_hbm, out_hbm, idx_v, rows_v, sem):
        wid = lax.axis_index("s") * 2 + lax.axis_index("c")
        base = wid * bpw
        pltpu.sync_copy(idx_hbm.at[pl.ds(base, bpw)], idx_v)
        pltpu.async_copy(table_hbm.at[idx_v], rows_v, sem).wait()
        pltpu.sync_copy(rows_v, out_hbm.at[pl.ds(base, bpw)])

    return k(table, idx)


def _sc_gather_rows2(table_a, table_b, idx):
    B = idx.shape[0]
    assert B % (8 * NW) == 0
    bpw = B // NW
    mesh = plsc.VectorSubcoreMesh(core_axis_name="c", subcore_axis_name="s")

    @functools.partial(
        pl.kernel,
        mesh=mesh,
        out_type=(jax.ShapeDtypeStruct((B, 16), table_a.dtype),
                  jax.ShapeDtypeStruct((B, 16), table_b.dtype)),
        compiler_params=pltpu.CompilerParams(use_tc_tiling_on_sc=False),
        scratch_types=[
            pltpu.VMEM((bpw,), jnp.int32),
            pltpu.VMEM((bpw, 16), table_a.dtype),
            pltpu.VMEM((bpw, 16), table_b.dtype),
            pltpu.SemaphoreType.DMA,
        ],
    )
    def k(ta_hbm, tb_hbm, idx_hbm, oa_hbm, ob_hbm, idx_v, ra_v, rb_v, sem):
        wid = lax.axis_index("s") * 2 + lax.axis_index("c")
        base = wid * bpw
        pltpu.sync_copy(idx_hbm.at[pl.ds(base, bpw)], idx_v)
        pltpu.async_copy(ta_hbm.at[idx_v], ra_v, sem).wait()
        pltpu.async_copy(tb_hbm.at[idx_v], rb_v, sem).wait()
        pltpu.sync_copy(ra_v, oa_hbm.at[pl.ds(base, bpw)])
        pltpu.sync_copy(rb_v, ob_hbm.at[pl.ds(base, bpw)])

    return k(table_a, table_b, idx)


def _centers_body(v0_ref, v1_ref, v2_ref, fc_ref, fd_ref):
    v0 = v0_ref[...]
    v1 = v1_ref[...]
    v2 = v2_ref[...]
    pid = pl.program_id(0)
    blk = v0.shape[0]
    fcs = []
    for c in range(3):
        s = _sum3_mean(v0[:, c:c + 1], v1[:, c:c + 1], v2[:, c:c + 1])
        fcs.append(s / 3.0)
    n2 = _sum3_sq(fcs[0] * fcs[0], fcs[1] * fcs[1], fcs[2] * fcs[2])
    rowid = lax.broadcasted_iota(jnp.int32, (blk, 1), 0) + pid * blk
    n2 = jnp.where(rowid >= N_F, jnp.float32(1e30), n2)
    fc_ref[...] = jnp.concatenate([fcs[0], fcs[1], fcs[2], n2], axis=1)
    dd = [(fcs[c] * 2.0).astype(jnp.bfloat16) for c in range(3)]
    fd_ref[...] = jnp.concatenate(
        dd + [jnp.zeros((blk, 13), jnp.bfloat16)], axis=1)


def _argmin_body(fc_ref, fd_ref, q_ref, qp_ref, fidx_ref, mind_ref, mask_ref):
    qb = q_ref[...]
    qx = qb[0:1, :]
    qy = qb[1:2, :]
    qz = qb[2:3, :]
    qn2 = _sum3_sq(qx * qx, qy * qy, qz * qz)
    qpb = qp_ref[...]
    iota_s = lax.broadcasted_iota(jnp.int32, (FBS, QB), 0)

    def step(b, carry):
        rmin, ridx = carry
        fn2 = fc_ref[pl.ds(b * FBS, FBS), 3:4]
        fdb = fd_ref[pl.ds(b * FBS, FBS), :]
        m2 = lax.dot_general(
            fdb, qpb, (((1,), (0,)), ((), ())),
            preferred_element_type=jnp.float32)
        d2 = (qn2 + fn2) - m2
        bmin = jnp.min(d2, axis=0, keepdims=True)
        cand = jnp.where(d2 == bmin, iota_s, jnp.int32(2 ** 30))
        barg = jnp.min(cand, axis=0, keepdims=True) + b * FBS
        upd = bmin < rmin
        return (jnp.where(upd, bmin, rmin), jnp.where(upd, barg, ridx))

    rmin0 = jnp.full((1, QB), jnp.inf, jnp.float32)
    ridx0 = jnp.zeros((1, QB), jnp.int32)
    rmin, ridx = lax.fori_loop(0, F_PAD // FBS, step, (rmin0, ridx0))
    fidx_ref[...] = ridx
    mind_ref[...] = rmin
    mask_ref[...] = (rmin > 0.1).astype(jnp.int32)


def _solver_body(tv_ref, tn_ref, q_ref, qn_ref, mind_ref, vw_ref):
    def col(ref, i):
        return ref[:, i:i + 1]

    T = [[col(tv_ref, 16 * k + c) for c in range(3)] for k in range(3)]
    N = [[col(tn_ref, 16 * k + c) for c in range(3)] for k in range(3)]
    q = [col(q_ref, c) for c in range(3)]
    qnr = [col(qn_ref, c) for c in range(3)]

    def dot3(a, b):
        return a[0] * b[0] + a[1] * b[1] + a[2] * b[2]

    nq = jnp.sqrt(dot3(qnr, qnr))
    qden = jnp.maximum(nq, 1e-12)
    qn = [c / qden for c in qnr]

    b0 = [T[0][c] - T[2][c] for c in range(3)]
    b1 = [T[1][c] - T[2][c] for c in range(3)]
    a0 = [N[0][c] - N[2][c] for c in range(3)]
    a1 = [N[1][c] - N[2][c] for c in range(3)]

    w0 = jnp.full_like(q[0], 1.0 / 3.0)
    w1 = jnp.full_like(q[0], 1.0 / 3.0)
    alpha = 1.0
    for _outer in range(2):
        d0 = jnp.zeros_like(w0)
        d1 = jnp.zeros_like(w0)
        A0 = jnp.zeros_like(w0)
        A1 = jnp.zeros_like(w0)
        for i in range(8):
            wd0 = w0 + d0
            wd1 = w1 + d1
            wd2 = (1.0 - wd0) - wd1
            cV = [T[0][c] * wd0 + T[1][c] * wd1 + T[2][c] * wd2 for c in range(3)]
            rv = [cV[c] - q[c] for c in range(3)]
            lv = jnp.sqrt(dot3(rv, rv))
            u = [N[0][c] * wd0 + N[1][c] * wd1 + N[2][c] * wd2 for c in range(3)]
            nu = jnp.sqrt(dot3(u, u))
            mden = jnp.maximum(nu, 1e-12)
            un = [u[c] / mden for c in range(3)]
            rn = [un[c] - qn[c] for c in range(3)]
            ln = jnp.sqrt(dot3(rn, rn))
            rn_u = dot3(rn, u)
            safe = nu > 1e-12
            inv_m2nu = jnp.where(safe, 1.0 / (nu * mden * mden), 0.0)
            gn0 = (dot3(rn, a0) / mden - rn_u * dot3(u, a0) * inv_m2nu) / ln
            gn1 = (dot3(rn, a1) / mden - rn_u * dot3(u, a1) * inv_m2nu) / ln
            g0 = (dot3(rv, b0) / lv + 0.01 * gn0) * (1.0 / 4096.0)
            g1 = (dot3(rv, b1) / lv + 0.01 * gn1) * (1.0 / 4096.0)
            A0 = A0 + g0 * g0
            A1 = A1 + g1 * g1
            clr = 0.2 / (1.0 + i * 0.1)
            d0 = d0 - clr * g0 / (jnp.sqrt(A0) + 1e-10)
            d1 = d1 - clr * g1 / (jnp.sqrt(A1) + 1e-10)
        w0 = w0 + d0 * alpha
        w1 = w1 + d1 * alpha
        alpha = alpha * 0.5

    outlier = mind_ref[...] > 0.1
    third = jnp.full_like(w0, 1.0 / 3.0)
    vw_ref[:, 0:1] = jnp.where(outlier, third, w0)
    vw_ref[:, 1:2] = jnp.where(outlier, third, w1)


def kernel(query_V, query_N, mesh_V, mesh_F, mesh_N, TC, FTC):
    b, n, _ = query_V.shape
    q = query_V.reshape(n, 3)
    qT = q.T
    qnT = query_N.reshape(n, 3).T

    mesh_F = mesh_F.astype(jnp.int32)
    mesh_V16 = jnp.pad(mesh_V, ((0, 0), (0, 13)))
    mesh_N16 = jnp.pad(mesh_N, ((0, 0), (0, 13)))
    mesh_F16 = jnp.pad(mesh_F, ((0, 0), (0, 13)))

    idxF = jnp.pad(mesh_F.T, ((0, 0), (0, F_PAD - N_F))).reshape(-1)
    vrows = _sc_gather_rows(mesh_V16, idxF)

    nb0 = F_PAD // 1024
    fc4, fd16 = pl.pallas_call(
        _centers_body,
        grid=(nb0,),
        in_specs=[
            pl.BlockSpec((1024, 16), lambda i: (i, 0)),
            pl.BlockSpec((1024, 16), lambda i: (i + nb0, 0)),
            pl.BlockSpec((1024, 16), lambda i: (i + 2 * nb0, 0)),
        ],
        out_specs=[
            pl.BlockSpec((1024, 4), lambda i: (i, 0)),
            pl.BlockSpec((1024, 16), lambda i: (i, 0)),
        ],
        out_shape=[
            jax.ShapeDtypeStruct((F_PAD, 4), jnp.float32),
            jax.ShapeDtypeStruct((F_PAD, 16), jnp.bfloat16),
        ],
    )(vrows, vrows, vrows)

    qp16 = jnp.pad(qT, ((0, 13), (0, 0))).astype(jnp.bfloat16)

    fidx2, mind2, mask2 = pl.pallas_call(
        _argmin_body,
        grid=(n // QB,),
        in_specs=[
            pl.BlockSpec((F_PAD, 4), lambda i: (0, 0)),
            pl.BlockSpec((F_PAD, 16), lambda i: (0, 0)),
            pl.BlockSpec((3, QB), lambda i: (0, i)),
            pl.BlockSpec((16, QB), lambda i: (0, i)),
        ],
        out_specs=[
            pl.BlockSpec((1, QB), lambda i: (0, i)),
            pl.BlockSpec((1, QB), lambda i: (0, i)),
            pl.BlockSpec((1, QB), lambda i: (0, i)),
        ],
        out_shape=[
            jax.ShapeDtypeStruct((1, n), jnp.int32),
            jax.ShapeDtypeStruct((1, n), jnp.float32),
            jax.ShapeDtypeStruct((1, n), jnp.int32),
        ],
    )(fc4, fd16, qT, qp16)

    frows = _sc_gather_rows(mesh_F16, fidx2.reshape(n))
    tri = frows[:, :3].reshape(-1)
    tvrows, tnrows = _sc_gather_rows2(mesh_V16, mesh_N16, tri)
    tv48 = tvrows.reshape(n, 48)
    tn48 = tnrows.reshape(n, 48)

    vw2 = pl.pallas_call(
        _solver_body,
        grid=(n // 512,),
        in_specs=[
            pl.BlockSpec((512, 48), lambda i: (i, 0)),
            pl.BlockSpec((512, 48), lambda i: (i, 0)),
            pl.BlockSpec((512, 3), lambda i: (i, 0)),
            pl.BlockSpec((512, 3), lambda i: (i, 0)),
            pl.BlockSpec((512, 1), lambda i: (i, 0)),
        ],
        out_specs=pl.BlockSpec((512, 2), lambda i: (i, 0)),
        out_shape=jax.ShapeDtypeStruct((n, 2), jnp.float32),
    )(tv48, tn48, q, query_N.reshape(n, 3), mind2.reshape(n, 1))

    spt_fidx = fidx2.reshape(b, n)
    spt_vw = vw2.reshape(b, n, 2)
    outlier = mask2.reshape(b, n).astype(jnp.bool_)
    return (spt_fidx, spt_vw, outlier)


# speedup vs baseline: 2.7946x; 2.7946x over previous
import functools

import jax
import jax.numpy as jnp
from jax import lax
from jax.experimental import pallas as pl
from jax.experimental.pallas import tpu as pltpu
from jax.experimental.pallas import tpu_sc as plsc

N_Q = 4096
N_F = 20000
F_PAD = 20480
NW = 32
FBS = 512
QB = 4096


def _sum3_mean(a, b, c):
    return (a + b) + c


def _sum3_sq(a, b, c):
    return (a + c) + b


def _sum3_dot(a, b, c):
    return (a + c) + b


def _bf(x):
    return x.astype(jnp.bfloat16).astype(jnp.float32)


def _sc_gather_rows(table, idx):
    B = idx.shape[0]
    assert B % (8 * NW) == 0
    bpw = B // NW
    mesh = plsc.VectorSubcoreMesh(core_axis_name="c", subcore_axis_name="s")

    @functools.partial(
        pl.kernel,
        mesh=mesh,
        out_type=jax.ShapeDtypeStruct((B, 16), table.dtype),
        compiler_params=pltpu.CompilerParams(use_tc_tiling_on_sc=False),
        scratch_types=[
            pltpu.VMEM((bpw,), jnp.int32),
            pltpu.VMEM((bpw, 16), table.dtype),
            pltpu.SemaphoreType.DMA,
        ],
    )
    def k(table_hbm, idx_hbm, out_hbm, idx_v, rows_v, sem):
        wid = lax.axis_index("s") * 2 + lax.axis_index("c")
        base = wid * bpw
        pltpu.sync_copy(idx_hbm.at[pl.ds(base, bpw)], idx_v)
        pltpu.async_copy(table_hbm.at[idx_v], rows_v, sem).wait()
        pltpu.sync_copy(rows_v, out_hbm.at[pl.ds(base, bpw)])

    return k(table, idx)


def _sc_gather_rows2(table_a, table_b, idx):
    B = idx.shape[0]
    assert B % (8 * NW) == 0
    bpw = B // NW
    mesh = plsc.VectorSubcoreMesh(core_axis_name="c", subcore_axis_name="s")

    @functools.partial(
        pl.kernel,
        mesh=mesh,
        out_type=(jax.ShapeDtypeStruct((B, 16), table_a.dtype),
                  jax.ShapeDtypeStruct((B, 16), table_b.dtype)),
        compiler_params=pltpu.CompilerParams(use_tc_tiling_on_sc=False),
        scratch_types=[
            pltpu.VMEM((bpw,), jnp.int32),
            pltpu.VMEM((bpw, 16), table_a.dtype),
            pltpu.VMEM((bpw, 16), table_b.dtype),
            pltpu.SemaphoreType.DMA,
        ],
    )
    def k(ta_hbm, tb_hbm, idx_hbm, oa_hbm, ob_hbm, idx_v, ra_v, rb_v, sem):
        wid = lax.axis_index("s") * 2 + lax.axis_index("c")
        base = wid * bpw
        pltpu.sync_copy(idx_hbm.at[pl.ds(base, bpw)], idx_v)
        pltpu.async_copy(ta_hbm.at[idx_v], ra_v, sem).wait()
        pltpu.async_copy(tb_hbm.at[idx_v], rb_v, sem).wait()
        pltpu.sync_copy(ra_v, oa_hbm.at[pl.ds(base, bpw)])
        pltpu.sync_copy(rb_v, ob_hbm.at[pl.ds(base, bpw)])

    return k(table_a, table_b, idx)


def _centers_body(v0_ref, v1_ref, v2_ref, fc_ref, fd_ref):
    v0 = v0_ref[...]
    v1 = v1_ref[...]
    v2 = v2_ref[...]
    pid = pl.program_id(0)
    blk = v0.shape[0]
    fcs = []
    for c in range(3):
        s = _sum3_mean(v0[:, c:c + 1], v1[:, c:c + 1], v2[:, c:c + 1])
        fcs.append(s / 3.0)
    n2 = _sum3_sq(fcs[0] * fcs[0], fcs[1] * fcs[1], fcs[2] * fcs[2])
    rowid = lax.broadcasted_iota(jnp.int32, (blk, 1), 0) + pid * blk
    n2 = jnp.where(rowid >= N_F, jnp.float32(1e30), n2)
    fc_ref[...] = jnp.concatenate([fcs[0], fcs[1], fcs[2], n2], axis=1)
    dd = [(fcs[c] * 2.0).astype(jnp.bfloat16) for c in range(3)]
    fd_ref[...] = jnp.concatenate(
        dd + [jnp.zeros((blk, 13), jnp.bfloat16)], axis=1)


def _argmin_body(fc_ref, fd_ref, q_ref, qp_ref, fidx_ref, mind_ref, mask_ref):
    qb = q_ref[...]
    qx = qb[0:1, :]
    qy = qb[1:2, :]
    qz = qb[2:3, :]
    qn2 = _sum3_sq(qx * qx, qy * qy, qz * qz)
    qpb = qp_ref[...]
    iota_s = lax.broadcasted_iota(jnp.int32, (FBS, QB), 0)

    def step(b, carry):
        rmin, ridx = carry
        fn2 = fc_ref[pl.ds(b * FBS, FBS), 3:4]
        fdb = fd_ref[pl.ds(b * FBS, FBS), :]
        m2 = lax.dot_general(
            fdb, qpb, (((1,), (0,)), ((), ())),
            preferred_element_type=jnp.float32)
        d2 = (qn2 + fn2) - m2
        bmin = jnp.min(d2, axis=0, keepdims=True)
        cand = jnp.where(d2 == bmin, iota_s, jnp.int32(2 ** 30))
        barg = jnp.min(cand, axis=0, keepdims=True) + b * FBS
        upd = bmin < rmin
        return (jnp.where(upd, bmin, rmin), jnp.where(upd, barg, ridx))

    rmin0 = jnp.full((1, QB), jnp.inf, jnp.float32)
    ridx0 = jnp.zeros((1, QB), jnp.int32)
    rmin, ridx = lax.fori_loop(0, F_PAD // FBS, step, (rmin0, ridx0))
    fidx_ref[...] = ridx
    mind_ref[...] = rmin
    mask_ref[...] = (rmin > 0.1).astype(jnp.int32)


def _solver_body(tv_ref, tn_ref, q_ref, qn_ref, mind_ref, vw_ref):
    def row(ref, i):
        return ref[i:i + 1, :]

    T = [[row(tv_ref, 3 * k + c) for c in range(3)] for k in range(3)]
    N = [[row(tn_ref, 3 * k + c) for c in range(3)] for k in range(3)]
    q = [row(q_ref, c) for c in range(3)]
    qnr = [row(qn_ref, c) for c in range(3)]

    def dot3(a, b):
        return a[0] * b[0] + a[1] * b[1] + a[2] * b[2]

    nq = jnp.sqrt(dot3(qnr, qnr))
    qden = jnp.maximum(nq, 1e-12)
    qn = [c / qden for c in qnr]

    b0 = [T[0][c] - T[2][c] for c in range(3)]
    b1 = [T[1][c] - T[2][c] for c in range(3)]
    a0 = [N[0][c] - N[2][c] for c in range(3)]
    a1 = [N[1][c] - N[2][c] for c in range(3)]

    w0 = jnp.full_like(q[0], 1.0 / 3.0)
    w1 = jnp.full_like(q[0], 1.0 / 3.0)
    alpha = 1.0
    for _outer in range(2):
        d0 = jnp.zeros_like(w0)
        d1 = jnp.zeros_like(w0)
        A0 = jnp.zeros_like(w0)
        A1 = jnp.zeros_like(w0)
        for i in range(8):
            wd0 = w0 + d0
            wd1 = w1 + d1
            wd2 = (1.0 - wd0) - wd1
            cV = [T[0][c] * wd0 + T[1][c] * wd1 + T[2][c] * wd2 for c in range(3)]
            rv = [cV[c] - q[c] for c in range(3)]
            lv = jnp.sqrt(dot3(rv, rv))
            u = [N[0][c] * wd0 + N[1][c] * wd1 + N[2][c] * wd2 for c in range(3)]
            nu = jnp.sqrt(dot3(u, u))
            mden = jnp.maximum(nu, 1e-12)
            un = [u[c] / mden for c in range(3)]
            rn = [un[c] - qn[c] for c in range(3)]
            ln = jnp.sqrt(dot3(rn, rn))
            rn_u = dot3(rn, u)
            safe = nu > 1e-12
            inv_m2nu = jnp.where(safe, 1.0 / (nu * mden * mden), 0.0)
            gn0 = (dot3(rn, a0) / mden - rn_u * dot3(u, a0) * inv_m2nu) / ln
            gn1 = (dot3(rn, a1) / mden - rn_u * dot3(u, a1) * inv_m2nu) / ln
            g0 = (dot3(rv, b0) / lv + 0.01 * gn0) * (1.0 / 4096.0)
            g1 = (dot3(rv, b1) / lv + 0.01 * gn1) * (1.0 / 4096.0)
            A0 = A0 + g0 * g0
            A1 = A1 + g1 * g1
            clr = 0.2 / (1.0 + i * 0.1)
            d0 = d0 - clr * g0 / (jnp.sqrt(A0) + 1e-10)
            d1 = d1 - clr * g1 / (jnp.sqrt(A1) + 1e-10)
        w0 = w0 + d0 * alpha
        w1 = w1 + d1 * alpha
        alpha = alpha * 0.5

    outlier = mind_ref[...] > 0.1
    third = jnp.full_like(w0, 1.0 / 3.0)
    vw_ref[0:1, :] = jnp.where(outlier, third, w0)
    vw_ref[1:2, :] = jnp.where(outlier, third, w1)


def kernel(query_V, query_N, mesh_V, mesh_F, mesh_N, TC, FTC):
    b, n, _ = query_V.shape
    q = query_V.reshape(n, 3)
    qT = q.T
    qnT = query_N.reshape(n, 3).T

    mesh_F = mesh_F.astype(jnp.int32)
    mesh_V16 = jnp.pad(mesh_V, ((0, 0), (0, 13)))
    mesh_N16 = jnp.pad(mesh_N, ((0, 0), (0, 13)))
    mesh_F16 = jnp.pad(mesh_F, ((0, 0), (0, 13)))

    idxF = jnp.pad(mesh_F.T, ((0, 0), (0, F_PAD - N_F))).reshape(-1)
    vrows = _sc_gather_rows(mesh_V16, idxF)

    nb0 = F_PAD // 1024
    fc4, fd16 = pl.pallas_call(
        _centers_body,
        grid=(nb0,),
        in_specs=[
            pl.BlockSpec((1024, 16), lambda i: (i, 0)),
            pl.BlockSpec((1024, 16), lambda i: (i + nb0, 0)),
            pl.BlockSpec((1024, 16), lambda i: (i + 2 * nb0, 0)),
        ],
        out_specs=[
            pl.BlockSpec((1024, 4), lambda i: (i, 0)),
            pl.BlockSpec((1024, 16), lambda i: (i, 0)),
        ],
        out_shape=[
            jax.ShapeDtypeStruct((F_PAD, 4), jnp.float32),
            jax.ShapeDtypeStruct((F_PAD, 16), jnp.bfloat16),
        ],
    )(vrows, vrows, vrows)

    qp16 = jnp.pad(qT, ((0, 13), (0, 0))).astype(jnp.bfloat16)

    fidx2, mind2, mask2 = pl.pallas_call(
        _argmin_body,
        grid=(n // QB,),
        in_specs=[
            pl.BlockSpec((F_PAD, 4), lambda i: (0, 0)),
            pl.BlockSpec((F_PAD, 16), lambda i: (0, 0)),
            pl.BlockSpec((3, QB), lambda i: (0, i)),
            pl.BlockSpec((16, QB), lambda i: (0, i)),
        ],
        out_specs=[
            pl.BlockSpec((1, QB), lambda i: (0, i)),
            pl.BlockSpec((1, QB), lambda i: (0, i)),
            pl.BlockSpec((1, QB), lambda i: (0, i)),
        ],
        out_shape=[
            jax.ShapeDtypeStruct((1, n), jnp.int32),
            jax.ShapeDtypeStruct((1, n), jnp.float32),
            jax.ShapeDtypeStruct((1, n), jnp.int32),
        ],
    )(fc4, fd16, qT, qp16)

    frows = _sc_gather_rows(mesh_F16, fidx2.reshape(n))
    tri = frows[:, :3].reshape(-1)
    tvrows, tnrows = _sc_gather_rows2(mesh_V16, mesh_N16, tri)
    tv9 = tvrows[:, :3].reshape(n, 3, 3).transpose(1, 2, 0).reshape(9, n)
    tn9 = tnrows[:, :3].reshape(n, 3, 3).transpose(1, 2, 0).reshape(9, n)

    vw2 = pl.pallas_call(
        _solver_body,
        grid=(n // 512,),
        in_specs=[
            pl.BlockSpec((9, 512), lambda i: (0, i)),
            pl.BlockSpec((9, 512), lambda i: (0, i)),
            pl.BlockSpec((3, 512), lambda i: (0, i)),
            pl.BlockSpec((3, 512), lambda i: (0, i)),
            pl.BlockSpec((1, 512), lambda i: (0, i)),
        ],
        out_specs=pl.BlockSpec((2, 512), lambda i: (0, i)),
        out_shape=jax.ShapeDtypeStruct((2, n), jnp.float32),
    )(tv9, tn9, qT, qnT, mind2)

    spt_fidx = fidx2.reshape(b, n)
    spt_vw = vw2.T.reshape(b, n, 2)
    outlier = mask2.reshape(b, n).astype(jnp.bool_)
    return (spt_fidx, spt_vw, outlier)
